# baseline (speedup 1.0000x reference)
import jax
import jax.numpy as jnp
from jax.experimental import pallas as pl
from jax.experimental.pallas import tpu as pltpu

_BB = 16
_FS = 256


def _fused_kernel(w_ref, b_ref, x_ref, o_ref):
    j = pl.program_id(1)
    w = w_ref[...]
    b = b_ref[...]
    for i in range(_BB):
        x = x_ref[i, :, pl.ds(j * _FS, _FS)].astype(jnp.bfloat16)
        acc = jnp.dot(w, x, preferred_element_type=jnp.float32)
        o_ref[i] = acc + b


def kernel(x, weight, bias):
    B, D, F = x.shape
    N = weight.shape[0]
    w_bf16 = weight.astype(jnp.bfloat16)
    bias_row = bias.reshape(1, F)

    return pl.pallas_call(
        _fused_kernel,
        out_shape=jax.ShapeDtypeStruct((B, N, F), x.dtype),
        grid=(B // _BB, F // _FS),
        in_specs=[
            pl.BlockSpec((N, D), lambda i, j: (0, 0)),
            pl.BlockSpec((1, _FS), lambda i, j: (0, j)),
            pl.BlockSpec((_BB, D, F), lambda i, j: (i, 0, 0)),
        ],
        out_specs=pl.BlockSpec((_BB, N, _FS), lambda i, j: (i, 0, j)),
        compiler_params=pltpu.CompilerParams(
            dimension_semantics=("parallel", "arbitrary"),
        ),
        cost_estimate=pl.CostEstimate(
            flops=2 * B * N * D * F,
            transcendentals=0,
            bytes_accessed=4 * (B * D * F + B * N * F) + 2 * N * D + 4 * F,
        ),
    )(w_bf16, bias_row, x)


# speedup vs baseline: 1.2459x; 1.2459x over previous
import jax
import jax.numpy as jnp
from jax.experimental import pallas as pl
from jax.experimental.pallas import tpu as pltpu


_BB = 8


def _fused_kernel(w_ref, b_ref, x0_ref, x1_ref, o_ref):
    w = w_ref[...]
    b = b_ref[...]
    h = _BB // 2
    for i in range(h):
        x = x0_ref[i].astype(jnp.bfloat16)
        acc = jnp.dot(w, x, preferred_element_type=jnp.float32)
        o_ref[i] = acc + b
    for i in range(h):
        x = x1_ref[i].astype(jnp.bfloat16)
        acc = jnp.dot(w, x, preferred_element_type=jnp.float32)
        o_ref[h + i] = acc + b


def kernel(x, weight, bias):
    B, D, F = x.shape
    N = weight.shape[0]
    w_bf16 = weight.astype(jnp.bfloat16)
    bias_row = bias.reshape(1, F)

    return pl.pallas_call(
        _fused_kernel,
        out_shape=jax.ShapeDtypeStruct((B, N, F), x.dtype),
        grid=(B // _BB,),
        in_specs=[
            pl.BlockSpec((N, D), lambda b: (0, 0)),
            pl.BlockSpec((1, F), lambda b: (0, 0)),
            pl.BlockSpec((_BB // 2, D, F), lambda b: (2 * b, 0, 0)),
            pl.BlockSpec((_BB // 2, D, F), lambda b: (2 * b + 1, 0, 0)),
        ],
        out_specs=pl.BlockSpec((_BB, N, F), lambda b: (b, 0, 0)),
        compiler_params=pltpu.CompilerParams(
            dimension_semantics=("parallel",),
        ),
        cost_estimate=pl.CostEstimate(
            flops=2 * B * N * D * F,
            transcendentals=0,
            bytes_accessed=4 * (B * D * F + B * N * F) + 2 * N * D + 4 * F,
        ),
    )(w_bf16, bias_row, x, x)


# speedup vs baseline: 1.2515x; 1.0045x over previous
import jax
import jax.numpy as jnp
from jax.experimental import pallas as pl
from jax.experimental.pallas import tpu as pltpu


_BB = 8


def _fused_kernel(w_ref, b_ref, x_ref, o_ref):
    w = w_ref[...]
    b = b_ref[...]
    for i in range(_BB):
        x = x_ref[i].astype(jnp.bfloat16)
        acc = jnp.dot(w, x, preferred_element_type=jnp.float32)
        o_ref[i] = acc + b


def kernel(x, weight, bias):
    B, D, F = x.shape
    N = weight.shape[0]
    w_bf16 = weight.astype(jnp.bfloat16)
    bias_row = bias.reshape(1, F)

    return pl.pallas_call(
        _fused_kernel,
        out_shape=jax.ShapeDtypeStruct((B, N, F), x.dtype),
        grid=(B // _BB,),
        in_specs=[
            pl.BlockSpec((N, D), lambda b: (0, 0)),
            pl.BlockSpec((1, F), lambda b: (0, 0)),
            pl.BlockSpec((_BB, D, F), lambda b: (b, 0, 0)),
        ],
        out_specs=pl.BlockSpec((_BB, N, F), lambda b: (b, 0, 0)),
        compiler_params=pltpu.CompilerParams(
            dimension_semantics=("parallel",),
        ),
        cost_estimate=pl.CostEstimate(
            flops=2 * B * N * D * F,
            transcendentals=0,
            bytes_accessed=4 * (B * D * F + B * N * F) + 2 * N * D + 4 * F,
        ),
    )(w_bf16, bias_row, x)
